# Initial kernel scaffold; baseline (speedup 1.0000x reference)
import jax
import jax.numpy as jnp
from jax.experimental import pallas as pl


def kernel(visual_embs, pos_table, seg_table):
    raise NotImplementedError("write your pallas kernel here")


# speedup vs baseline: 22.4333x; 22.4333x over previous
import jax
import jax.numpy as jnp
from jax.experimental import pallas as pl


def _body(v_ref, p_ref, s_ref, o_ref):
    bias = p_ref[...] + s_ref[0, :][None, :]
    o_ref[...] = v_ref[...] + bias[None, :, :]


def kernel(visual_embs, pos_table, seg_table):
    bsz, length, dim = visual_embs.shape
    BB = 512
    grid = (bsz // BB,)
    return pl.pallas_call(
        _body,
        grid=grid,
        in_specs=[
            pl.BlockSpec((BB, length, dim), lambda i: (i, 0, 0)),
            pl.BlockSpec((length, dim), lambda i: (0, 0)),
            pl.BlockSpec((2, dim), lambda i: (0, 0)),
        ],
        out_specs=pl.BlockSpec((BB, length, dim), lambda i: (i, 0, 0)),
        out_shape=jax.ShapeDtypeStruct(visual_embs.shape, visual_embs.dtype),
    )(visual_embs, pos_table, seg_table)
